# baseline (speedup 1.0000x reference)
import functools

import jax
import jax.numpy as jnp
from jax import lax
from jax.experimental import pallas as pl
from jax.experimental.pallas import tpu as pltpu
from jax.experimental.pallas import tpu_sc as plsc

_B = 4096
_H = 200
_D = 64
_C = 128

_NC = 2
_NS = 16
_NW = _NC * _NS
_BPW = _B // _NW

_G0 = 128
_G1 = _H - _G0

_DL = _D // 16


def _pool_body(x_hbm, table_hbm, out_hbm, idx_v, rows_v, acc_v, sem):
    c = lax.axis_index("c")
    s = lax.axis_index("s")
    wid = s * _NC + c
    base = wid * _BPW

    pltpu.sync_copy(x_hbm.at[pl.ds(base * _H, _BPW * _H)], idx_v)

    def batch_body(b, _):
        off = b * _H
        cp0 = pltpu.async_copy(
            table_hbm.at[idx_v.at[pl.ds(off, _G0)]],
            rows_v.at[pl.ds(0, _G0)], sem)
        cp1 = pltpu.async_copy(
            table_hbm.at[idx_v.at[pl.ds(off + _G0, _G1)]],
            rows_v.at[pl.ds(_G0, _G1)], sem)
        cp0.wait()
        cp1.wait()

        def row_body(r, accs):
            return tuple(accs[d] + rows_v[r, pl.ds(d * 16, 16)]
                         for d in range(_DL))

        accs = tuple(jnp.zeros((16,), jnp.float32) for _ in range(_DL))
        accs = lax.fori_loop(0, _H, row_body, accs)
        for d in range(_DL):
            acc_v[b, pl.ds(d * 16, 16)] = accs[d]
        return 0

    lax.fori_loop(0, _BPW, batch_body, 0)
    pltpu.sync_copy(acc_v, out_hbm.at[pl.ds(base, _BPW)])


@functools.partial(jax.jit, static_argnums=())
def _pool(x_flat, table):
    mesh = plsc.VectorSubcoreMesh(core_axis_name="c", subcore_axis_name="s",
                                  num_cores=_NC, num_subcores=_NS)
    f = pl.kernel(
        _pool_body,
        out_type=jax.ShapeDtypeStruct((_B, _D), jnp.float32),
        mesh=mesh,
        compiler_params=pltpu.CompilerParams(use_tc_tiling_on_sc=False),
        scratch_types=[
            pltpu.VMEM((_BPW * _H,), jnp.int32),
            pltpu.VMEM((_H, _D), jnp.float32),
            pltpu.VMEM((_BPW, _D), jnp.float32),
            pltpu.SemaphoreType.DMA,
        ],
    )
    return f(x_flat, table)


def _fc_body(pooled_ref, inv_len_ref, w_ref, b_ref, out_ref):
    pooled = pooled_ref[...] * inv_len_ref[...]
    out = lax.dot_general(pooled, w_ref[...], (((1,), (1,)), ((), ())),
                          preferred_element_type=jnp.float32)
    out_ref[...] = out + b_ref[...]


@jax.jit
def _fc(pooled, x_len, fc_w, fc_b):
    inv_len = (1.0 / x_len.astype(jnp.float32)).reshape(_B, 1)
    return pl.pallas_call(
        _fc_body,
        out_shape=jax.ShapeDtypeStruct((_B, _C), jnp.float32),
    )(pooled, inv_len, fc_w, fc_b.reshape(1, _C))


def kernel(x, x_len, table, fc_w, fc_b):
    pooled = _pool(x.reshape(-1), table)
    return _fc(pooled, x_len, fc_w, fc_b)


# speedup vs baseline: 1.1687x; 1.1687x over previous
import functools

import jax
import jax.numpy as jnp
from jax import lax
from jax.experimental import pallas as pl
from jax.experimental.pallas import tpu as pltpu
from jax.experimental.pallas import tpu_sc as plsc

_B = 4096
_H = 200
_D = 64
_C = 128

_NC = 2
_NS = 16
_NW = _NC * _NS
_BPW = _B // _NW

_G0 = 128
_G1 = _H - _G0

_DL = _D // 16


_NBUF = 2


def _pool_body(x_hbm, table_hbm, out_hbm, idx_v, rows_v, acc_v, sem0, sem1):
    c = lax.axis_index("c")
    s = lax.axis_index("s")
    wid = s * _NC + c
    base = wid * _BPW
    sems = (sem0, sem1)

    pltpu.sync_copy(x_hbm.at[pl.ds(base * _H, _BPW * _H)], idx_v)

    def fire(b, p):
        off = b * _H
        pltpu.async_copy(
            table_hbm.at[idx_v.at[pl.ds(off, _G0)]],
            rows_v.at[p, pl.ds(0, _G0)], sems[p])
        pltpu.async_copy(
            table_hbm.at[idx_v.at[pl.ds(off + _G0, _G1)]],
            rows_v.at[p, pl.ds(_G0, _G1)], sems[p])

    def drain(p):
        pltpu.make_async_copy(
            table_hbm.at[pl.ds(0, _H)], rows_v.at[p], sems[p]).wait()

    def accumulate(b, p):
        def row_body(r, accs):
            return tuple(accs[d] + rows_v[p, r, pl.ds(d * 16, 16)]
                         for d in range(_DL))

        accs = tuple(jnp.zeros((16,), jnp.float32) for _ in range(_DL))
        accs = pl.loop(0, _H, init_carry=accs, unroll=4)(row_body)
        for d in range(_DL):
            acc_v[b, pl.ds(d * 16, 16)] = accs[d]

    for p in range(_NBUF):
        fire(p, p)

    @pl.loop(0, _BPW - _NBUF, step=_NBUF)
    def _(bb):
        for p in range(_NBUF):
            drain(p)
            accumulate(bb + p, p)
            fire(bb + p + _NBUF, p)

    for p in range(_NBUF):
        drain(p)
        accumulate(_BPW - _NBUF + p, p)

    pltpu.sync_copy(acc_v, out_hbm.at[pl.ds(base, _BPW)])


@functools.partial(jax.jit, static_argnums=())
def _pool(x_flat, table):
    mesh = plsc.VectorSubcoreMesh(core_axis_name="c", subcore_axis_name="s",
                                  num_cores=_NC, num_subcores=_NS)
    f = pl.kernel(
        _pool_body,
        out_type=jax.ShapeDtypeStruct((_B, _D), jnp.float32),
        mesh=mesh,
        compiler_params=pltpu.CompilerParams(use_tc_tiling_on_sc=False),
        scratch_types=[
            pltpu.VMEM((_BPW * _H,), jnp.int32),
            pltpu.VMEM((_NBUF, _H, _D), jnp.float32),
            pltpu.VMEM((_BPW, _D), jnp.float32),
            pltpu.SemaphoreType.DMA,
            pltpu.SemaphoreType.DMA,
        ],
    )
    return f(x_flat, table)


def _fc_body(pooled_ref, inv_len_ref, w_ref, b_ref, out_ref):
    pooled = pooled_ref[...] * inv_len_ref[...]
    out = lax.dot_general(pooled, w_ref[...], (((1,), (1,)), ((), ())),
                          preferred_element_type=jnp.float32)
    out_ref[...] = out + b_ref[...]


@jax.jit
def _fc(pooled, x_len, fc_w, fc_b):
    inv_len = (1.0 / x_len.astype(jnp.float32)).reshape(_B, 1)
    return pl.pallas_call(
        _fc_body,
        out_shape=jax.ShapeDtypeStruct((_B, _C), jnp.float32),
    )(pooled, inv_len, fc_w, fc_b.reshape(1, _C))


def kernel(x, x_len, table, fc_w, fc_b):
    pooled = _pool(x.reshape(-1), table)
    return _fc(pooled, x_len, fc_w, fc_b)


# speedup vs baseline: 1.5205x; 1.3011x over previous
import functools

import jax
import jax.numpy as jnp
from jax import lax
from jax.experimental import pallas as pl
from jax.experimental.pallas import tpu as pltpu
from jax.experimental.pallas import tpu_sc as plsc

_B = 4096
_H = 200
_D = 64
_C = 128
_V = 1000000

_NC = 2
_NS = 16
_NW = _NC * _NS
_BPW = _B // _NW

_G0 = 128
_G1 = _H - _G0

_CL = _C // 16
_NBUF = 2

_VB = 4096


def _mm_body(tt_ref, wt_ref, g_ref):
    g_ref[...] = lax.dot_general(
        tt_ref[...], wt_ref[...], (((0,), (0,)), ((), ())),
        preferred_element_type=jnp.float32)


@jax.jit
def _mm(table_t, fc_wt):
    grid = (_V + _VB - 1) // _VB
    return pl.pallas_call(
        _mm_body,
        grid=(grid,),
        in_specs=[
            pl.BlockSpec((_D, _VB), lambda i: (0, i)),
            pl.BlockSpec((_D, _C), lambda i: (0, 0)),
        ],
        out_specs=pl.BlockSpec((_VB, _C), lambda i: (i, 0)),
        out_shape=jax.ShapeDtypeStruct((_V, _C), jnp.float32),
    )(table_t, fc_wt)


def _pool_body(x_hbm, g_hbm, out_hbm, idx_v, rows_v, acc_v, sem0, sem1):
    c = lax.axis_index("c")
    s = lax.axis_index("s")
    wid = s * _NC + c
    base = wid * _BPW
    sems = (sem0, sem1)

    pltpu.sync_copy(x_hbm.at[pl.ds(base * _H, _BPW * _H)], idx_v)

    def fire(b, p):
        off = b * _H
        pltpu.async_copy(
            g_hbm.at[idx_v.at[pl.ds(off, _G0)]],
            rows_v.at[p, pl.ds(0, _G0)], sems[p])
        pltpu.async_copy(
            g_hbm.at[idx_v.at[pl.ds(off + _G0, _G1)]],
            rows_v.at[p, pl.ds(_G0, _G1)], sems[p])

    def drain(p):
        pltpu.make_async_copy(
            g_hbm.at[pl.ds(0, _H)], rows_v.at[p], sems[p]).wait()

    def accumulate(b, p):
        def row_body(r, accs):
            return tuple(accs[d] + rows_v[p, r, pl.ds(d * 16, 16)]
                         for d in range(_CL))

        accs = tuple(jnp.zeros((16,), jnp.float32) for _ in range(_CL))
        accs = pl.loop(0, _H, init_carry=accs, unroll=4)(row_body)
        for d in range(_CL):
            acc_v[b, pl.ds(d * 16, 16)] = accs[d]

    for p in range(_NBUF):
        fire(p, p)

    @pl.loop(0, _BPW - _NBUF, step=_NBUF)
    def _(bb):
        for p in range(_NBUF):
            drain(p)
            accumulate(bb + p, p)
            fire(bb + p + _NBUF, p)

    for p in range(_NBUF):
        drain(p)
        accumulate(_BPW - _NBUF + p, p)

    pltpu.sync_copy(acc_v, out_hbm.at[pl.ds(base, _BPW)])


@jax.jit
def _pool(x_flat, g):
    mesh = plsc.VectorSubcoreMesh(core_axis_name="c", subcore_axis_name="s",
                                  num_cores=_NC, num_subcores=_NS)
    f = pl.kernel(
        _pool_body,
        out_type=jax.ShapeDtypeStruct((_B, _C), jnp.float32),
        mesh=mesh,
        compiler_params=pltpu.CompilerParams(use_tc_tiling_on_sc=True),
        scratch_types=[
            pltpu.VMEM((_BPW * _H,), jnp.int32),
            pltpu.VMEM((_NBUF, _H, _C), jnp.float32),
            pltpu.VMEM((_BPW, _C), jnp.float32),
            pltpu.SemaphoreType.DMA,
            pltpu.SemaphoreType.DMA,
        ],
    )
    return f(x_flat, g)


def _eps_body(acc_ref, len_ref, b_ref, out_ref):
    inv = 1.0 / len_ref[...].astype(jnp.float32)
    out_ref[...] = acc_ref[...] * inv + b_ref[...]


@jax.jit
def _eps(acc, x_len, fc_b):
    return pl.pallas_call(
        _eps_body,
        out_shape=jax.ShapeDtypeStruct((_B, _C), jnp.float32),
    )(acc, x_len.reshape(_B, 1), fc_b.reshape(1, _C))


def kernel(x, x_len, table, fc_w, fc_b):
    g = _mm(table.T, fc_w.T)
    acc = _pool(x.reshape(-1), g)
    return _eps(acc, x_len, fc_b)


# speedup vs baseline: 1.7654x; 1.1610x over previous
import functools

import jax
import jax.numpy as jnp
from jax import lax
from jax.experimental import pallas as pl
from jax.experimental.pallas import tpu as pltpu
from jax.experimental.pallas import tpu_sc as plsc

_B = 4096
_H = 200
_D = 64
_C = 128
_V = 1000000

_NC = 2
_NS = 16
_NW = _NC * _NS
_BPW = _B // _NW

_G0 = 128
_G1 = _H - _G0

_CL = _C // 16
_NBUF = 2

_VB = 8192


def _mm_body(tt_ref, wt_ref, g_ref):
    g_ref[...] = lax.dot_general(
        tt_ref[...], wt_ref[...], (((0,), (0,)), ((), ())),
        preferred_element_type=jnp.float32)


@jax.jit
def _mm(table_t, fc_wt):
    grid = (_V + _VB - 1) // _VB
    return pl.pallas_call(
        _mm_body,
        grid=(grid,),
        in_specs=[
            pl.BlockSpec((_D, _VB), lambda i: (0, i)),
            pl.BlockSpec((_D, _C), lambda i: (0, 0)),
        ],
        out_specs=pl.BlockSpec((_VB, _C), lambda i: (i, 0)),
        out_shape=jax.ShapeDtypeStruct((_V, _C), jnp.float32),
    )(table_t, fc_wt)


def _pool_body(x_hbm, g_hbm, out_hbm, idx_v, rows_v, acc_v, sem0, sem1):
    c = lax.axis_index("c")
    s = lax.axis_index("s")
    wid = s * _NC + c
    base = wid * _BPW
    sems = (sem0, sem1)

    pltpu.sync_copy(x_hbm.at[pl.ds(base * _H, _BPW * _H)], idx_v)

    def fire(b, p):
        off = b * _H
        pltpu.async_copy(
            g_hbm.at[idx_v.at[pl.ds(off, _G0)]],
            rows_v.at[p, pl.ds(0, _G0)], sems[p])
        pltpu.async_copy(
            g_hbm.at[idx_v.at[pl.ds(off + _G0, _G1)]],
            rows_v.at[p, pl.ds(_G0, _G1)], sems[p])

    def drain(p):
        pltpu.make_async_copy(
            g_hbm.at[pl.ds(0, _H)], rows_v.at[p], sems[p]).wait()

    def accumulate(b, p):
        def row_body(r, accs):
            return tuple(accs[d] + rows_v[p, r, pl.ds(d * 16, 16)]
                         for d in range(_CL))

        accs = tuple(jnp.zeros((16,), jnp.float32) for _ in range(_CL))
        accs = pl.loop(0, _H, init_carry=accs, unroll=8)(row_body)
        for d in range(_CL):
            acc_v[b, pl.ds(d * 16, 16)] = accs[d]

    for p in range(_NBUF):
        fire(p, p)

    @pl.loop(0, _BPW - _NBUF, step=_NBUF)
    def _(bb):
        for p in range(_NBUF):
            drain(p)
            accumulate(bb + p, p)
            fire(bb + p + _NBUF, p)

    for p in range(_NBUF):
        drain(p)
        accumulate(_BPW - _NBUF + p, p)

    pltpu.sync_copy(acc_v, out_hbm.at[pl.ds(base, _BPW)])


@jax.jit
def _pool(x_flat, g):
    mesh = plsc.VectorSubcoreMesh(core_axis_name="c", subcore_axis_name="s",
                                  num_cores=_NC, num_subcores=_NS)
    f = pl.kernel(
        _pool_body,
        out_type=jax.ShapeDtypeStruct((_B, _C), jnp.float32),
        mesh=mesh,
        compiler_params=pltpu.CompilerParams(use_tc_tiling_on_sc=True),
        scratch_types=[
            pltpu.VMEM((_BPW * _H,), jnp.int32),
            pltpu.VMEM((_NBUF, _H, _C), jnp.float32),
            pltpu.VMEM((_BPW, _C), jnp.float32),
            pltpu.SemaphoreType.DMA,
            pltpu.SemaphoreType.DMA,
        ],
    )
    return f(x_flat, g)


def _eps_body(acc_ref, len_ref, b_ref, out_ref):
    inv = 1.0 / len_ref[...].astype(jnp.float32)
    out_ref[...] = acc_ref[...] * inv + b_ref[...]


@jax.jit
def _eps(acc, x_len, fc_b):
    return pl.pallas_call(
        _eps_body,
        out_shape=jax.ShapeDtypeStruct((_B, _C), jnp.float32),
    )(acc, x_len.reshape(_B, 1), fc_b.reshape(1, _C))


def kernel(x, x_len, table, fc_w, fc_b):
    g = _mm(table.T, fc_w.T)
    acc = _pool(x.reshape(-1), g)
    return _eps(acc, x_len, fc_b)


# speedup vs baseline: 1.8644x; 1.0561x over previous
import functools

import jax
import jax.numpy as jnp
from jax import lax
from jax.experimental import pallas as pl
from jax.experimental.pallas import tpu as pltpu
from jax.experimental.pallas import tpu_sc as plsc

_B = 4096
_H = 200
_D = 64
_C = 128
_V = 1000000

_NC = 2
_NS = 16
_NW = _NC * _NS
_BPW = _B // _NW

_G0 = 128
_G1 = _H - _G0

_CL = _C // 16
_NBUF = 2

_VB = 16384


def _mm_body(tt_ref, wt_ref, g_ref):
    g_ref[...] = lax.dot_general(
        tt_ref[...], wt_ref[...], (((0,), (0,)), ((), ())),
        precision=lax.Precision.DEFAULT,
        preferred_element_type=jnp.float32)


@jax.jit
def _mm(table_t, fc_wt):
    grid = (_V + _VB - 1) // _VB
    return pl.pallas_call(
        _mm_body,
        grid=(grid,),
        in_specs=[
            pl.BlockSpec((_D, _VB), lambda i: (0, i)),
            pl.BlockSpec((_D, _C), lambda i: (0, 0)),
        ],
        out_specs=pl.BlockSpec((_VB, _C), lambda i: (i, 0)),
        out_shape=jax.ShapeDtypeStruct((_V, _C), jnp.float32),
    )(table_t, fc_wt)


def _pool_body(x_hbm, g_hbm, out_hbm, idx_v, rows_v, acc_v, sem0, sem1):
    c = lax.axis_index("c")
    s = lax.axis_index("s")
    wid = s * _NC + c
    base = wid * _BPW
    sems = (sem0, sem1)

    pltpu.sync_copy(x_hbm.at[pl.ds(base * _H, _BPW * _H)], idx_v)

    def fire(b, p):
        off = b * _H
        pltpu.async_copy(
            g_hbm.at[idx_v.at[pl.ds(off, _G0)]],
            rows_v.at[p, pl.ds(0, _G0)], sems[p])
        pltpu.async_copy(
            g_hbm.at[idx_v.at[pl.ds(off + _G0, _G1)]],
            rows_v.at[p, pl.ds(_G0, _G1)], sems[p])

    def drain(p):
        pltpu.make_async_copy(
            g_hbm.at[pl.ds(0, _H)], rows_v.at[p], sems[p]).wait()

    def accumulate(b, p):
        def row_body(r, accs):
            return tuple(accs[d] + rows_v[p, r, pl.ds(d * 16, 16)]
                         for d in range(_CL))

        accs = tuple(jnp.zeros((16,), jnp.float32) for _ in range(_CL))
        accs = pl.loop(0, _H, init_carry=accs, unroll=8)(row_body)
        for d in range(_CL):
            acc_v[b, pl.ds(d * 16, 16)] = accs[d]

    for p in range(_NBUF):
        fire(p, p)

    @pl.loop(0, _BPW - _NBUF, step=_NBUF)
    def _(bb):
        for p in range(_NBUF):
            drain(p)
            accumulate(bb + p, p)
            fire(bb + p + _NBUF, p)

    for p in range(_NBUF):
        drain(p)
        accumulate(_BPW - _NBUF + p, p)

    pltpu.sync_copy(acc_v, out_hbm.at[pl.ds(base, _BPW)])


@jax.jit
def _pool(x_flat, g):
    mesh = plsc.VectorSubcoreMesh(core_axis_name="c", subcore_axis_name="s",
                                  num_cores=_NC, num_subcores=_NS)
    f = pl.kernel(
        _pool_body,
        out_type=jax.ShapeDtypeStruct((_B, _C), jnp.float32),
        mesh=mesh,
        compiler_params=pltpu.CompilerParams(use_tc_tiling_on_sc=True),
        scratch_types=[
            pltpu.VMEM((_BPW * _H,), jnp.int32),
            pltpu.VMEM((_NBUF, _H, _C), jnp.float32),
            pltpu.VMEM((_BPW, _C), jnp.float32),
            pltpu.SemaphoreType.DMA,
            pltpu.SemaphoreType.DMA,
        ],
    )
    return f(x_flat, g)


def _eps_body(acc_ref, len_ref, b_ref, out_ref):
    inv = 1.0 / len_ref[...].astype(jnp.float32)
    out_ref[...] = acc_ref[...] * inv + b_ref[...]


@jax.jit
def _eps(acc, x_len, fc_b):
    return pl.pallas_call(
        _eps_body,
        out_shape=jax.ShapeDtypeStruct((_B, _C), jnp.float32),
    )(acc, x_len.reshape(_B, 1), fc_b.reshape(1, _C))


def kernel(x, x_len, table, fc_w, fc_b):
    g = _mm(table.T, fc_w.T)
    acc = _pool(x.reshape(-1), g)
    return _eps(acc, x_len, fc_b)


# speedup vs baseline: 2.0272x; 1.0873x over previous
import functools

import jax
import jax.numpy as jnp
from jax import lax
from jax.experimental import pallas as pl
from jax.experimental.pallas import tpu as pltpu
from jax.experimental.pallas import tpu_sc as plsc

_B = 4096
_H = 200
_D = 64
_C = 128
_V = 1000000

_NC = 2
_NS = 16
_NW = _NC * _NS
_BPW = _B // _NW

_HB0 = 104
_HB1 = _H - _HB0

_CL = _C // 16
_NBUF = 4
_NJ = 2 * _BPW

_VB = 16384


def _mm_body(tt_ref, wt_ref, g_ref):
    g_ref[...] = lax.dot_general(
        tt_ref[...], wt_ref[...], (((0,), (0,)), ((), ())),
        precision=lax.Precision.DEFAULT,
        preferred_element_type=jnp.float32)


@jax.jit
def _mm(table_t, fc_wt):
    grid = (_V + _VB - 1) // _VB
    return pl.pallas_call(
        _mm_body,
        grid=(grid,),
        in_specs=[
            pl.BlockSpec((_D, _VB), lambda i: (0, i)),
            pl.BlockSpec((_D, _C), lambda i: (0, 0)),
        ],
        out_specs=pl.BlockSpec((_VB, _C), lambda i: (i, 0)),
        out_shape=jax.ShapeDtypeStruct((_V, _C), jnp.float32),
    )(table_t, fc_wt)


def _pool_body(x_hbm, g_hbm, out_hbm, idx_v, rows_v, acc_v,
               sem0, sem1, sem2, sem3):
    c = lax.axis_index("c")
    s = lax.axis_index("s")
    wid = s * _NC + c
    base = wid * _BPW
    sems = (sem0, sem1, sem2, sem3)

    pltpu.sync_copy(x_hbm.at[pl.ds(base * _H, _BPW * _H)], idx_v)

    def fire(j, p, half):
        off = (j // 2) * _H + (_HB0 if half else 0)
        sz = _HB1 if half else _HB0
        pltpu.async_copy(
            g_hbm.at[idx_v.at[pl.ds(off, sz)]],
            rows_v.at[p, pl.ds(0, sz)], sems[p])

    def drain(p, half):
        sz = _HB1 if half else _HB0
        pltpu.make_async_copy(
            g_hbm.at[pl.ds(0, sz)], rows_v.at[p, pl.ds(0, sz)],
            sems[p]).wait()

    def add_rows(accs, p, half):
        sz = _HB1 if half else _HB0

        def row_body(r, accs):
            return tuple(accs[d] + rows_v[p, r, pl.ds(d * 16, 16)]
                         for d in range(_CL))

        return pl.loop(0, sz, init_carry=accs, unroll=8)(row_body)

    def zeros():
        return tuple(jnp.zeros((16,), jnp.float32) for _ in range(_CL))

    def process(j, p, fire_next):
        half = p & 1
        drain(p, half)
        if half == 0:
            process.accs = add_rows(zeros(), p, 0)
        else:
            accs = add_rows(process.accs, p, 1)
            b = j // 2
            for d in range(_CL):
                acc_v[b, pl.ds(d * 16, 16)] = accs[d]
        if fire_next:
            fire(j + _NBUF, p, half)

    for p in range(_NBUF):
        fire(p, p, p & 1)

    @pl.loop(0, _NJ - _NBUF, step=_NBUF)
    def _(jj):
        for p in range(_NBUF):
            process(jj + p, p, True)

    for p in range(_NBUF):
        process(_NJ - _NBUF + p, p, False)

    pltpu.sync_copy(acc_v, out_hbm.at[pl.ds(base, _BPW)])


@jax.jit
def _pool(x_flat, g):
    mesh = plsc.VectorSubcoreMesh(core_axis_name="c", subcore_axis_name="s",
                                  num_cores=_NC, num_subcores=_NS)
    f = pl.kernel(
        _pool_body,
        out_type=jax.ShapeDtypeStruct((_B, _C), jnp.float32),
        mesh=mesh,
        compiler_params=pltpu.CompilerParams(use_tc_tiling_on_sc=True),
        scratch_types=[
            pltpu.VMEM((_BPW * _H,), jnp.int32),
            pltpu.VMEM((_NBUF, _HB0, _C), jnp.float32),
            pltpu.VMEM((_BPW, _C), jnp.float32),
            pltpu.SemaphoreType.DMA,
            pltpu.SemaphoreType.DMA,
            pltpu.SemaphoreType.DMA,
            pltpu.SemaphoreType.DMA,
        ],
    )
    return f(x_flat, g)


def _eps_body(acc_ref, len_ref, b_ref, out_ref):
    inv = 1.0 / len_ref[...].astype(jnp.float32)
    out_ref[...] = acc_ref[...] * inv + b_ref[...]


@jax.jit
def _eps(acc, x_len, fc_b):
    return pl.pallas_call(
        _eps_body,
        out_shape=jax.ShapeDtypeStruct((_B, _C), jnp.float32),
    )(acc, x_len.reshape(_B, 1), fc_b.reshape(1, _C))


def kernel(x, x_len, table, fc_w, fc_b):
    g = _mm(table.T, fc_w.T)
    acc = _pool(x.reshape(-1), g)
    return _eps(acc, x_len, fc_b)


# speedup vs baseline: 2.0626x; 1.0175x over previous
import functools

import jax
import jax.numpy as jnp
from jax import lax
from jax.experimental import pallas as pl
from jax.experimental.pallas import tpu as pltpu
from jax.experimental.pallas import tpu_sc as plsc

_B = 4096
_H = 200
_D = 64
_C = 128
_V = 1000000

_NC = 2
_NS = 16
_NW = _NC * _NS
_BPW = _B // _NW

_HB0 = 104
_HB1 = _H - _HB0

_CL = _C // 16
_NBUF = 4
_NJ = 2 * _BPW

_VB = 32768


def _mm_body(tt_ref, wt_ref, g_ref):
    g_ref[...] = lax.dot_general(
        tt_ref[...], wt_ref[...], (((0,), (0,)), ((), ())),
        precision=lax.Precision.DEFAULT,
        preferred_element_type=jnp.float32)


@jax.jit
def _mm(table_t, fc_wt):
    grid = (_V + _VB - 1) // _VB
    return pl.pallas_call(
        _mm_body,
        grid=(grid,),
        in_specs=[
            pl.BlockSpec((_D, _VB), lambda i: (0, i)),
            pl.BlockSpec((_D, _C), lambda i: (0, 0)),
        ],
        out_specs=pl.BlockSpec((_VB, _C), lambda i: (i, 0)),
        out_shape=jax.ShapeDtypeStruct((_V, _C), jnp.float32),
    )(table_t, fc_wt)


def _pool_body(x_hbm, g_hbm, out_hbm, idx_v, rows_v, acc_v,
               sem0, sem1, sem2, sem3):
    c = lax.axis_index("c")
    s = lax.axis_index("s")
    wid = s * _NC + c
    base = wid * _BPW
    sems = (sem0, sem1, sem2, sem3)

    pltpu.sync_copy(x_hbm.at[pl.ds(base * _H, _BPW * _H)], idx_v)

    def fire(j, p, half):
        off = (j // 2) * _H + (_HB0 if half else 0)
        sz = _HB1 if half else _HB0
        pltpu.async_copy(
            g_hbm.at[idx_v.at[pl.ds(off, sz)]],
            rows_v.at[p, pl.ds(0, sz)], sems[p])

    def drain(p, half):
        sz = _HB1 if half else _HB0
        pltpu.make_async_copy(
            g_hbm.at[pl.ds(0, sz)], rows_v.at[p, pl.ds(0, sz)],
            sems[p]).wait()

    def add_rows(accs, p, half):
        sz = _HB1 if half else _HB0

        def row_body(r, accs):
            return tuple(accs[d] + rows_v[p, r, pl.ds(d * 16, 16)]
                         for d in range(_CL))

        return pl.loop(0, sz, init_carry=accs, unroll=8)(row_body)

    def zeros():
        return tuple(jnp.zeros((16,), jnp.float32) for _ in range(_CL))

    def process(j, p, fire_next):
        half = p & 1
        drain(p, half)
        if half == 0:
            process.accs = add_rows(zeros(), p, 0)
        else:
            accs = add_rows(process.accs, p, 1)
            b = j // 2
            for d in range(_CL):
                acc_v[b, pl.ds(d * 16, 16)] = accs[d]
        if fire_next:
            fire(j + _NBUF, p, half)

    for p in range(_NBUF):
        fire(p, p, p & 1)

    @pl.loop(0, _NJ - _NBUF, step=_NBUF)
    def _(jj):
        for p in range(_NBUF):
            process(jj + p, p, True)

    for p in range(_NBUF):
        process(_NJ - _NBUF + p, p, False)

    pltpu.sync_copy(acc_v, out_hbm.at[pl.ds(base, _BPW)])


@jax.jit
def _pool(x_flat, g):
    mesh = plsc.VectorSubcoreMesh(core_axis_name="c", subcore_axis_name="s",
                                  num_cores=_NC, num_subcores=_NS)
    f = pl.kernel(
        _pool_body,
        out_type=jax.ShapeDtypeStruct((_B, _C), jnp.float32),
        mesh=mesh,
        compiler_params=pltpu.CompilerParams(use_tc_tiling_on_sc=True),
        scratch_types=[
            pltpu.VMEM((_BPW * _H,), jnp.int32),
            pltpu.VMEM((_NBUF, _HB0, _C), jnp.float32),
            pltpu.VMEM((_BPW, _C), jnp.float32),
            pltpu.SemaphoreType.DMA,
            pltpu.SemaphoreType.DMA,
            pltpu.SemaphoreType.DMA,
            pltpu.SemaphoreType.DMA,
        ],
    )
    return f(x_flat, g)


def _eps_body(acc_ref, len_ref, b_ref, out_ref):
    inv = 1.0 / len_ref[...].astype(jnp.float32)
    out_ref[...] = acc_ref[...] * inv + b_ref[...]


@jax.jit
def _eps(acc, x_len, fc_b):
    return pl.pallas_call(
        _eps_body,
        out_shape=jax.ShapeDtypeStruct((_B, _C), jnp.float32),
    )(acc, x_len.reshape(_B, 1), fc_b.reshape(1, _C))


def kernel(x, x_len, table, fc_w, fc_b):
    g = _mm(table.T, fc_w.T)
    acc = _pool(x.reshape(-1), g)
    return _eps(acc, x_len, fc_b)


# speedup vs baseline: 2.1029x; 1.0195x over previous
import functools

import jax
import jax.numpy as jnp
from jax import lax
from jax.experimental import pallas as pl
from jax.experimental.pallas import tpu as pltpu
from jax.experimental.pallas import tpu_sc as plsc

_B = 4096
_H = 200
_D = 64
_C = 128
_V = 1000000

_NC = 2
_NS = 16
_NW = _NC * _NS
_BPW = _B // _NW

_HB0 = 104
_HB1 = _H - _HB0

_CL = _C // 16
_NBUF = 4
_NJ = 2 * _BPW

_VB = 32768


def _mm_body(tt_ref, wt_ref, g_ref):
    g_ref[...] = lax.dot_general(
        tt_ref[...], wt_ref[...], (((0,), (0,)), ((), ())),
        precision=lax.Precision.DEFAULT,
        preferred_element_type=jnp.float32)


@jax.jit
def _mm(table_t, fc_wt):
    grid = (_V + _VB - 1) // _VB
    return pl.pallas_call(
        _mm_body,
        grid=(grid,),
        in_specs=[
            pl.BlockSpec((_D, _VB), lambda i: (0, i)),
            pl.BlockSpec((_D, _C), lambda i: (0, 0)),
        ],
        out_specs=pl.BlockSpec((_VB, _C), lambda i: (i, 0)),
        out_shape=jax.ShapeDtypeStruct((_V, _C), jnp.float32),
    )(table_t, fc_wt)


def _pool_body(xt_hbm, g_hbm, len_hbm, b_hbm, out_hbm,
               xb_v, idx_v, rows_v, acc_v, len_v, bias_v,
               sem0, sem1, sem2, sem3):
    c = lax.axis_index("c")
    s = lax.axis_index("s")
    wid = s * _NC + c
    base = wid * _BPW
    sems = (sem0, sem1, sem2, sem3)

    pltpu.sync_copy(xt_hbm.at[:, pl.ds(base, _BPW)], xb_v)
    pltpu.sync_copy(len_hbm.at[pl.ds(base, _BPW)], len_v)
    pltpu.sync_copy(b_hbm, bias_v)

    lanes = lax.iota(jnp.int32, 16)
    addr0 = tuple(( (j * 16 + lanes) * _H ) for j in range(_BPW // 16))

    @pl.loop(0, _H)
    def _(l):
        for j in range(_BPW // 16):
            plsc.store_scatter(idx_v, [addr0[j] + l],
                               xb_v[l, pl.ds(j * 16, 16)])

    def fire(j, p, half):
        off = (j // 2) * _H + (_HB0 if half else 0)
        sz = _HB1 if half else _HB0
        pltpu.async_copy(
            g_hbm.at[idx_v.at[pl.ds(off, sz)]],
            rows_v.at[p, pl.ds(0, sz)], sems[p])

    def drain(p, half):
        sz = _HB1 if half else _HB0
        pltpu.make_async_copy(
            g_hbm.at[pl.ds(0, sz)], rows_v.at[p, pl.ds(0, sz)],
            sems[p]).wait()

    def add_rows(accs, p, half):
        sz = _HB1 if half else _HB0

        def row_body(r, accs):
            return tuple(accs[d] + rows_v[p, r, pl.ds(d * 16, 16)]
                         for d in range(_CL))

        return pl.loop(0, sz, init_carry=accs, unroll=8)(row_body)

    def zeros():
        return tuple(jnp.zeros((16,), jnp.float32) for _ in range(_CL))

    def process(j, p, fire_next):
        half = p & 1
        drain(p, half)
        if half == 0:
            process.accs = add_rows(zeros(), p, 0)
        else:
            accs = add_rows(process.accs, p, 1)
            b = j // 2
            lvec = plsc.load_gather(len_v, [jnp.zeros((16,), jnp.int32) + b])
            inv = 1.0 / lvec.astype(jnp.float32)
            for d in range(_CL):
                acc_v[b, pl.ds(d * 16, 16)] = (
                    accs[d] * inv + bias_v[pl.ds(d * 16, 16)])
        if fire_next:
            fire(j + _NBUF, p, half)

    for p in range(_NBUF):
        fire(p, p, p & 1)

    @pl.loop(0, _NJ - _NBUF, step=_NBUF)
    def _(jj):
        for p in range(_NBUF):
            process(jj + p, p, True)

    for p in range(_NBUF):
        process(_NJ - _NBUF + p, p, False)

    pltpu.sync_copy(acc_v, out_hbm.at[pl.ds(base, _BPW)])


@jax.jit
def _pool(xt, g, x_len, fc_b):
    mesh = plsc.VectorSubcoreMesh(core_axis_name="c", subcore_axis_name="s",
                                  num_cores=_NC, num_subcores=_NS)
    f = pl.kernel(
        _pool_body,
        out_type=jax.ShapeDtypeStruct((_B, _C), jnp.float32),
        mesh=mesh,
        compiler_params=pltpu.CompilerParams(use_tc_tiling_on_sc=True,
                                             needs_layout_passes=False),
        scratch_types=[
            pltpu.VMEM((_H, _BPW), jnp.int32),
            pltpu.VMEM((_BPW * _H,), jnp.int32),
            pltpu.VMEM((_NBUF, _HB0, _C), jnp.float32),
            pltpu.VMEM((_BPW, _C), jnp.float32),
            pltpu.VMEM((_BPW,), jnp.int32),
            pltpu.VMEM((_C,), jnp.float32),
            pltpu.SemaphoreType.DMA,
            pltpu.SemaphoreType.DMA,
            pltpu.SemaphoreType.DMA,
            pltpu.SemaphoreType.DMA,
        ],
    )
    return f(xt, g, x_len, fc_b)


def kernel(x, x_len, table, fc_w, fc_b):
    g = _mm(table.T, fc_w.T)
    return _pool(x.T, g, x_len, fc_b)


# speedup vs baseline: 2.1037x; 1.0004x over previous
import jax
import jax.numpy as jnp
from jax import lax
from jax.experimental import pallas as pl
from jax.experimental.pallas import tpu as pltpu
from jax.experimental.pallas import tpu_sc as plsc

_B = 4096
_H = 200
_D = 64
_C = 128
_V = 1000000

_NC = 2
_NS = 16
_NW = _NC * _NS
_BPW = _B // _NW

_HB0 = 104
_HB1 = _H - _HB0

_CL = _C // 16
_NBUF = 4
_NJ = 2 * _BPW

_VB = 32768


def _mm_body(tt_ref, wt_ref, g_ref):
    g_ref[...] = lax.dot_general(
        tt_ref[...], wt_ref[...], (((0,), (0,)), ((), ())),
        precision=lax.Precision.DEFAULT,
        preferred_element_type=jnp.float32)


@jax.jit
def _mm(table_t, fc_wt):
    grid = (_V + _VB - 1) // _VB
    return pl.pallas_call(
        _mm_body,
        grid=(grid,),
        in_specs=[
            pl.BlockSpec((_D, _VB), lambda i: (0, i)),
            pl.BlockSpec((_D, _C), lambda i: (0, 0)),
        ],
        out_specs=pl.BlockSpec((_VB, _C), lambda i: (i, 0)),
        out_shape=jax.ShapeDtypeStruct((_V, _C), jnp.float32),
    )(table_t, fc_wt)


def _pool_body(xt_hbm, g_hbm, len_hbm, b_hbm, out_hbm,
               xb_v, idx_v, rows_v, acc_v, len_v, bias_v,
               sem0, sem1, sem2, sem3):
    c = lax.axis_index("c")
    s = lax.axis_index("s")
    wid = s * _NC + c
    base = wid * _BPW
    sems = (sem0, sem1, sem2, sem3)

    pltpu.sync_copy(xt_hbm.at[:, pl.ds(base, _BPW)], xb_v)
    pltpu.sync_copy(len_hbm.at[pl.ds(base, _BPW)], len_v)
    pltpu.sync_copy(b_hbm, bias_v)

    lanes = lax.iota(jnp.int32, 16)
    addr0 = tuple(( (j * 16 + lanes) * _H ) for j in range(_BPW // 16))

    @pl.loop(0, _H)
    def _(l):
        for j in range(_BPW // 16):
            plsc.store_scatter(idx_v, [addr0[j] + l],
                               xb_v[l, pl.ds(j * 16, 16)])

    def fire(j, p, half):
        off = (j // 2) * _H + (_HB0 if half else 0)
        sz = _HB1 if half else _HB0
        pltpu.async_copy(
            g_hbm.at[idx_v.at[pl.ds(off, sz)]],
            rows_v.at[p, pl.ds(0, sz)], sems[p])

    def drain(p, half):
        sz = _HB1 if half else _HB0
        pltpu.make_async_copy(
            g_hbm.at[pl.ds(0, sz)], rows_v.at[p, pl.ds(0, sz)],
            sems[p]).wait()

    def add_rows(accs, p, half):
        sz = _HB1 if half else _HB0

        def row_body(r, accs):
            return tuple(accs[d] + rows_v[p, r, pl.ds(d * 16, 16)]
                         for d in range(_CL))

        return pl.loop(0, sz, init_carry=accs, unroll=8)(row_body)

    def zeros():
        return tuple(jnp.zeros((16,), jnp.float32) for _ in range(_CL))

    def process(j, p, fire_next):
        half = p & 1
        drain(p, half)
        if half == 0:
            process.accs = add_rows(zeros(), p, 0)
        else:
            accs = add_rows(process.accs, p, 1)
            b = j // 2
            lvec = plsc.load_gather(len_v, [jnp.zeros((16,), jnp.int32) + b])
            inv = 1.0 / lvec.astype(jnp.float32)
            for d in range(_CL):
                acc_v[b, pl.ds(d * 16, 16)] = (
                    accs[d] * inv + bias_v[pl.ds(d * 16, 16)])
        if fire_next:
            fire(j + _NBUF, p, half)

    for p in range(_NBUF):
        fire(p, p, p & 1)

    @pl.loop(0, _NJ - _NBUF, step=_NBUF)
    def _(jj):
        for p in range(_NBUF):
            process(jj + p, p, True)

    for p in range(_NBUF):
        process(_NJ - _NBUF + p, p, False)

    pltpu.sync_copy(acc_v, out_hbm.at[pl.ds(base, _BPW)])


@jax.jit
def _pool(xt, g, x_len, fc_b):
    mesh = plsc.VectorSubcoreMesh(core_axis_name="c", subcore_axis_name="s",
                                  num_cores=_NC, num_subcores=_NS)
    f = pl.kernel(
        _pool_body,
        out_type=jax.ShapeDtypeStruct((_B, _C), jnp.float32),
        mesh=mesh,
        compiler_params=pltpu.CompilerParams(use_tc_tiling_on_sc=True,
                                             needs_layout_passes=False),
        scratch_types=[
            pltpu.VMEM((_H, _BPW), jnp.int32),
            pltpu.VMEM((_BPW * _H,), jnp.int32),
            pltpu.VMEM((_NBUF, _HB0, _C), jnp.float32),
            pltpu.VMEM((_BPW, _C), jnp.float32),
            pltpu.VMEM((_BPW,), jnp.int32),
            pltpu.VMEM((_C,), jnp.float32),
            pltpu.SemaphoreType.DMA,
            pltpu.SemaphoreType.DMA,
            pltpu.SemaphoreType.DMA,
            pltpu.SemaphoreType.DMA,
        ],
    )
    return f(xt, g, x_len, fc_b)


def kernel(x, x_len, table, fc_w, fc_b):
    g = _mm(table.T, fc_w.T)
    return _pool(x.T, g, x_len, fc_b)
